# baseline (speedup 1.0000x reference)
import functools

import jax
import jax.numpy as jnp
from jax import lax
from jax.experimental import pallas as pl
from jax.experimental.pallas import tpu as pltpu
from jax.experimental.pallas import tpu_sc as plsc

_B = 16384
_D = 64
_NIDX = 2 * _B

_info = plsc.get_sparse_core_info()
_NC, _NS = _info.num_cores, _info.num_subcores
_NW = _NC * _NS
_IPW = _NIDX // _NW
_CHUNK = 128
_NCHUNK = _IPW // _CHUNK


def _sc_gather(idx_all, table2):
    mesh = plsc.VectorSubcoreMesh(core_axis_name="c", subcore_axis_name="s")

    @functools.partial(
        pl.kernel,
        mesh=mesh,
        out_type=jax.ShapeDtypeStruct((_NIDX, 2 * _D), jnp.float32),
        scratch_types=[
            pltpu.VMEM((_IPW,), jnp.int32),
            pltpu.VMEM((_CHUNK, 2 * _D), jnp.float32),
            pltpu.VMEM((_CHUNK, 2 * _D), jnp.float32),
            pltpu.SemaphoreType.DMA,
            pltpu.SemaphoreType.DMA,
        ],
    )
    def gather_k(table_hbm, idx_hbm, out_hbm, idx_v, buf0, buf1, sem0, sem1):
        wid = lax.axis_index("s") * _NC + lax.axis_index("c")
        base = wid * _IPW
        pltpu.sync_copy(idx_hbm.at[pl.ds(base, _IPW)], idx_v)
        for i in range(_IPW // 16):
            sl = pl.ds(i * 16, 16)
            idx_v[sl] = lax.shift_right_logical(idx_v[sl], 1)
        bufs = (buf0, buf1)
        sems = (sem0, sem1)
        cp = pltpu.async_copy(
            table_hbm.at[idx_v.at[pl.ds(0, _CHUNK)]], bufs[0], sems[0])
        for k in range(1, _NCHUNK):
            nxt = pltpu.async_copy(
                table_hbm.at[idx_v.at[pl.ds(k * _CHUNK, _CHUNK)]],
                bufs[k % 2], sems[k % 2])
            cp.wait()
            pltpu.sync_copy(
                bufs[(k - 1) % 2],
                out_hbm.at[pl.ds(base + (k - 1) * _CHUNK, _CHUNK)])
            cp = nxt
        cp.wait()
        pltpu.sync_copy(
            bufs[(_NCHUNK - 1) % 2],
            out_hbm.at[pl.ds(base + (_NCHUNK - 1) * _CHUNK, _CHUNK)])

    return gather_k(table2, idx_all)


def _tc_body(es_ref, eo_ref, sj_ref, ob_ref, r_ref, out_ref):
    es2 = es_ref[...]
    eo2 = eo_ref[...]
    e_s = jnp.where((sj_ref[...] & 1) == 1, es2[:, _D:], es2[:, :_D])
    e_o = jnp.where((ob_ref[...] & 1) == 1, eo2[:, _D:], eo2[:, :_D])
    t = lax.dot_general(e_o, r_ref[...],
                        (((1,), (1,)), ((), ())),
                        preferred_element_type=jnp.float32)
    s = jnp.sum(e_s * t, axis=-1, keepdims=True)
    out_ref[...] = jax.nn.sigmoid(s)


def _tc_score(rows2, subject, obj, relation):
    blk = 2048
    grid = (_B // blk,)
    out = pl.pallas_call(
        _tc_body,
        grid=grid,
        in_specs=[
            pl.BlockSpec((blk, 2 * _D), lambda i: (i, 0)),
            pl.BlockSpec((blk, 2 * _D), lambda i: (i + _B // blk, 0)),
            pl.BlockSpec((blk, 1), lambda i: (i, 0)),
            pl.BlockSpec((blk, 1), lambda i: (i, 0)),
            pl.BlockSpec((_D, _D), lambda i: (0, 0)),
        ],
        out_specs=pl.BlockSpec((blk, 1), lambda i: (i, 0)),
        out_shape=jax.ShapeDtypeStruct((_B, 1), jnp.float32),
    )(rows2, rows2, subject.reshape(_B, 1), obj.reshape(_B, 1), relation)
    return out.reshape(_B)


@jax.jit
def kernel(subject, obj, object_embeddings, relation):
    table2 = object_embeddings.reshape(-1, 2 * _D)
    idx_all = jnp.concatenate([subject, obj]).astype(jnp.int32)
    rows2 = _sc_gather(idx_all, table2)
    return _tc_score(rows2, subject, obj, relation)


# speedup vs baseline: 1.0005x; 1.0005x over previous
import functools

import jax
import jax.numpy as jnp
from jax import lax
from jax.experimental import pallas as pl
from jax.experimental.pallas import tpu as pltpu
from jax.experimental.pallas import tpu_sc as plsc

_B = 16384
_D = 64
_V = 1000000
_NIDX = 2 * _B

_info = plsc.get_sparse_core_info()
_NC, _NS = _info.num_cores, _info.num_subcores
_NW = _NC * _NS
_IPW = _NIDX // _NW
_CHUNK = 128
_NCHUNK = _IPW // _CHUNK


def _sc_gather(idx_all, table2):
    mesh = plsc.VectorSubcoreMesh(core_axis_name="c", subcore_axis_name="s")

    @functools.partial(
        pl.kernel,
        mesh=mesh,
        out_type=jax.ShapeDtypeStruct((_NIDX, 2 * _D), jnp.float32),
        scratch_types=[
            pltpu.VMEM((_IPW,), jnp.int32),
            pltpu.VMEM((_CHUNK, 2 * _D), jnp.float32),
            pltpu.VMEM((_CHUNK, 2 * _D), jnp.float32),
            pltpu.SemaphoreType.DMA,
            pltpu.SemaphoreType.DMA,
        ],
    )
    def gather_k(table_hbm, idx_hbm, out_hbm, idx_v, buf0, buf1, sem0, sem1):
        wid = lax.axis_index("s") * _NC + lax.axis_index("c")
        base = wid * _IPW
        pltpu.sync_copy(idx_hbm.at[pl.ds(base, _IPW)], idx_v)
        for i in range(_IPW // 16):
            sl = pl.ds(i * 16, 16)
            idx_v[sl] = lax.shift_right_logical(idx_v[sl], 1)
        bufs = (buf0, buf1)
        sems = (sem0, sem1)
        cp = pltpu.async_copy(
            table_hbm.at[idx_v.at[pl.ds(0, _CHUNK)]], bufs[0], sems[0])
        for k in range(1, _NCHUNK):
            nxt = pltpu.async_copy(
                table_hbm.at[idx_v.at[pl.ds(k * _CHUNK, _CHUNK)]],
                bufs[k % 2], sems[k % 2])
            cp.wait()
            pltpu.sync_copy(
                bufs[(k - 1) % 2],
                out_hbm.at[pl.ds(base + (k - 1) * _CHUNK, _CHUNK)])
            cp = nxt
        cp.wait()
        pltpu.sync_copy(
            bufs[(_NCHUNK - 1) % 2],
            out_hbm.at[pl.ds(base + (_NCHUNK - 1) * _CHUNK, _CHUNK)])

    return gather_k(table2, idx_all)


def _tc_body(es_ref, eo_ref, sj_ref, ob_ref, r_ref, out_ref):
    es2 = es_ref[...]
    eo2 = eo_ref[...]
    e_s = jnp.where((sj_ref[...] & 1) == 1, es2[:, _D:], es2[:, :_D])
    e_o = jnp.where((ob_ref[...] & 1) == 1, eo2[:, _D:], eo2[:, :_D])
    t = lax.dot_general(e_o, r_ref[...],
                        (((1,), (1,)), ((), ())),
                        preferred_element_type=jnp.float32)
    s = jnp.sum(e_s * t, axis=-1, keepdims=True)
    out_ref[...] = jax.nn.sigmoid(s)


def _tc_score(rows2, subject, obj, relation):
    blk = 2048
    grid = (_B // blk,)
    out = pl.pallas_call(
        _tc_body,
        grid=grid,
        in_specs=[
            pl.BlockSpec((blk, 2 * _D), lambda i: (i, 0)),
            pl.BlockSpec((blk, 2 * _D), lambda i: (i + _B // blk, 0)),
            pl.BlockSpec((blk, 1), lambda i: (i, 0)),
            pl.BlockSpec((blk, 1), lambda i: (i, 0)),
            pl.BlockSpec((_D, _D), lambda i: (0, 0)),
        ],
        out_specs=pl.BlockSpec((blk, 1), lambda i: (i, 0)),
        out_shape=jax.ShapeDtypeStruct((_B, 1), jnp.float32),
    )(rows2, rows2, subject.reshape(_B, 1), obj.reshape(_B, 1), relation)
    return out.reshape(_B)


@jax.jit
def kernel(subject, obj, object_embeddings, relation):
    table2 = object_embeddings.reshape(_V // 2, 2 * _D)
    idx_all = jnp.concatenate([subject, obj]).astype(jnp.int32)
    rows2 = _sc_gather(idx_all, table2)
    return _tc_score(rows2, subject, obj, relation)
